# Initial kernel scaffold; baseline (speedup 1.0000x reference)
import jax
import jax.numpy as jnp
from jax.experimental import pallas as pl


def kernel(z, emb):
    raise NotImplementedError("write your pallas kernel here")


# speedup vs baseline: 1.4090x; 1.4090x over previous
import functools

import jax
import jax.numpy as jnp
from jax import lax
from jax.experimental import pallas as pl
from jax.experimental.pallas import tpu as pltpu
from jax.experimental.pallas import tpu_sc as plsc

TOK_BLK = 512
K_TILE = 2048
N_CODES = 8192
C = 32


def _argmin_body(z_ref, embT_ref, idx_ref):
    z = z_ref[...]
    zb = z.astype(jnp.bfloat16)
    z2 = jnp.sum(z * z, axis=1, keepdims=True)
    half_v = []
    half_i = []
    for h in range(2):
        minval = jnp.full((TOK_BLK, 1), jnp.inf, dtype=jnp.float32)
        minidx = jnp.zeros((TOK_BLK, 1), dtype=jnp.int32)
        for t in range(N_CODES // (2 * K_TILE)):
            j = h * (N_CODES // (2 * K_TILE)) + t
            w = embT_ref[:, j * K_TILE:(j + 1) * K_TILE]
            e2 = jnp.sum(w * w, axis=0, keepdims=True)
            p = lax.dot_general(zb, w.astype(jnp.bfloat16),
                                (((1,), (0,)), ((), ())),
                                preferred_element_type=jnp.float32)
            d = (z2 - 2.0 * p) + e2
            m = jnp.min(d, axis=1, keepdims=True)
            iota = lax.broadcasted_iota(jnp.int32, (TOK_BLK, K_TILE), 1)
            li = jnp.min(jnp.where(d == m, iota + j * K_TILE,
                                   jnp.int32(2 ** 30)),
                         axis=1, keepdims=True)
            better = m < minval
            minval = jnp.where(better, m, minval)
            minidx = jnp.where(better, li, minidx)
        half_v.append(minval)
        half_i.append(minidx)
    m0_bf = half_v[0].astype(jnp.bfloat16).astype(jnp.float32)
    take1 = half_v[1] < m0_bf
    idx_ref[...] = jnp.where(take1, half_i[1], half_i[0])


def _tc_argmin(zf, embT):
    n = zf.shape[0]
    grid = n // TOK_BLK
    return pl.pallas_call(
        _argmin_body,
        grid=(grid,),
        in_specs=[
            pl.BlockSpec((TOK_BLK, C), lambda i: (i, 0)),
            pl.BlockSpec((C, N_CODES), lambda i: (0, 0)),
        ],
        out_specs=pl.BlockSpec((TOK_BLK, 1), lambda i: (i, 0)),
        out_shape=jax.ShapeDtypeStruct((n, 1), jnp.int32),
    )(zf, embT)


def _sc_gather(emb, idx3, n_tokens):
    info = plsc.get_sparse_core_info()
    nw = info.num_cores * info.num_subcores
    chunks, chunk = idx3.shape[1], idx3.shape[2]
    b_per_w = chunks * chunk
    mesh = plsc.VectorSubcoreMesh(core_axis_name="c", subcore_axis_name="s")

    @functools.partial(
        pl.kernel,
        mesh=mesh,
        out_type=jax.ShapeDtypeStruct((n_tokens, C), jnp.float32),
        scratch_types=[
            pltpu.VMEM((chunks, chunk), jnp.int32),
            pltpu.VMEM((b_per_w, C), jnp.float32),
            pltpu.SemaphoreType.DMA,
        ],
        compiler_params=pltpu.CompilerParams(use_tc_tiling_on_sc=False),
    )
    def gather_kernel(emb_hbm, idx_hbm, out_hbm, idx_v, rows_v, sem):
        wid = lax.axis_index("s") * info.num_cores + lax.axis_index("c")
        pltpu.sync_copy(idx_hbm.at[wid], idx_v)
        handles = [
            pltpu.async_copy(emb_hbm.at[idx_v.at[j]],
                             rows_v.at[pl.ds(j * chunk, chunk)], sem)
            for j in range(chunks)
        ]
        for hdl in handles:
            hdl.wait()
        pltpu.sync_copy(rows_v, out_hbm.at[pl.ds(wid * b_per_w, b_per_w)])

    return gather_kernel(emb, idx3)


def kernel(z, emb):
    b, c, h, w = z.shape
    n = b * h * w
    zf = jnp.transpose(z, (0, 2, 3, 1)).reshape(n, c).astype(jnp.float32)
    embT = emb.astype(jnp.float32).T
    idx = _tc_argmin(zf, embT)
    info = plsc.get_sparse_core_info()
    nw = info.num_cores * info.num_subcores
    idx3 = idx.reshape(nw, (n // nw) // 128, 128)
    zq = _sc_gather(emb.astype(jnp.float32), idx3, n)
    zq = zq.reshape(b, h, w, c)
    zq = jnp.transpose(zq, (0, 3, 1, 2))
    return zq.astype(z.dtype)
